# baseline (speedup 1.0000x reference)
import jax
import jax.numpy as jnp
import numpy as np
from jax.experimental import pallas as pl
from jax.experimental.pallas import tpu as pltpu

_B, _CIN, _H, _W = 4, 256, 56, 56
_COUT, _K = 256, 3
_KK = _K * _K
_HW = _H * _W
_TILE_P = 448
_NT = _HW // _TILE_P


def _dc_kernel(xt_ref, owr_ref, offb_ref, wr_ref, rw_ref, wcol_ref, yx_ref,
               ey_ref, ex_ref, out_ref, offs_ref):
    t = pl.program_id(1)
    x = xt_ref[0]

    @pl.when(t == 0)
    def _compute_offsets():
        wcol = wcol_ref[...]
        acc = jnp.zeros((_HW, 2 * _KK), jnp.float32) + offb_ref[...]
        for j in range(_KK):
            dy, dx = j // 3 - 1, j % 3 - 1
            s = dy * _W + dx
            if s > 0:
                xs = jnp.concatenate(
                    [x[s:], jnp.zeros((s, _CIN), jnp.float32)], axis=0)
            elif s < 0:
                xs = jnp.concatenate(
                    [jnp.zeros((-s, _CIN), jnp.float32), x[:_HW + s]], axis=0)
            else:
                xs = x
            if dx == 1:
                xs = jnp.where(wcol < _W - 1.5, xs, 0.0)
            elif dx == -1:
                xs = jnp.where(wcol > 0.5, xs, 0.0)
            acc = acc + jnp.dot(xs, owr_ref[j],
                                preferred_element_type=jnp.float32)
        offs_ref[...] = acc

    off_t = offs_ref[pl.ds(t * _TILE_P, _TILE_P), :]
    r = rw_ref[:, 0:1]
    w = rw_ref[:, 1:2]
    lane = yx_ref[0:1, :]
    ey = ey_ref[...]
    ex = ex_ref[...]

    acc = jnp.zeros((_TILE_P, _COUT), jnp.float32)
    for kk in range(_KK):
        kh, kw = kk // 3, kk % 3
        py = r + (kh - 1) + off_t[:, 2 * kk:2 * kk + 1]
        px = w + (kw - 1) + off_t[:, 2 * kk + 1:2 * kk + 2]
        wy_s = jnp.maximum(0.0, 1.0 - jnp.abs(lane - py))
        wx_s = jnp.maximum(0.0, 1.0 - jnp.abs(lane - px))
        wy = jnp.dot(wy_s, ey, preferred_element_type=jnp.float32)
        wx = jnp.dot(wx_s, ex, preferred_element_type=jnp.float32)
        samp = jnp.dot(wy * wx, x, preferred_element_type=jnp.float32)
        acc = acc + jnp.dot(samp, wr_ref[kk],
                            preferred_element_type=jnp.float32)
    out_ref[0] = acc


@jax.jit
def kernel(x, weight, off_w, off_b):
    B, C, H, W = x.shape
    xt = jnp.transpose(x, (0, 2, 3, 1)).reshape(B, _HW, C)
    owr = jnp.transpose(off_w, (2, 3, 1, 0)).reshape(_KK, C, 2 * _KK)
    offb2 = off_b.reshape(1, 2 * _KK)
    wr = jnp.transpose(weight.reshape(_COUT, C, _KK), (2, 1, 0))

    pv = np.arange(_HW)
    rw = jnp.asarray(np.stack([pv // _W, pv % _W], axis=1), jnp.float32)
    wcol = jnp.asarray((pv % _W)[:, None], jnp.float32)
    lane = jnp.asarray(np.arange(64)[None, :], jnp.float32)
    ey_np = (np.arange(64)[:, None] == (pv // _W)[None, :]).astype(np.float32)
    ex_np = (np.arange(64)[:, None] == (pv % _W)[None, :]).astype(np.float32)
    ey = jnp.asarray(ey_np)
    ex = jnp.asarray(ex_np)

    out = pl.pallas_call(
        _dc_kernel,
        grid=(B, _NT),
        in_specs=[
            pl.BlockSpec((1, _HW, C), lambda b, t: (b, 0, 0)),
            pl.BlockSpec((_KK, C, 2 * _KK), lambda b, t: (0, 0, 0)),
            pl.BlockSpec((1, 2 * _KK), lambda b, t: (0, 0)),
            pl.BlockSpec((_KK, C, _COUT), lambda b, t: (0, 0, 0)),
            pl.BlockSpec((_TILE_P, 2), lambda b, t: (t, 0)),
            pl.BlockSpec((_HW, 1), lambda b, t: (0, 0)),
            pl.BlockSpec((1, 64), lambda b, t: (0, 0)),
            pl.BlockSpec((64, _HW), lambda b, t: (0, 0)),
            pl.BlockSpec((64, _HW), lambda b, t: (0, 0)),
        ],
        out_specs=pl.BlockSpec((1, _TILE_P, _COUT), lambda b, t: (b, t, 0)),
        out_shape=jax.ShapeDtypeStruct((B, _HW, _COUT), jnp.float32),
        scratch_shapes=[pltpu.VMEM((_HW, 2 * _KK), jnp.float32)],
        compiler_params=pltpu.CompilerParams(
            dimension_semantics=("parallel", "arbitrary"),
        ),
    )(xt, owr, offb2, wr, rw, wcol, lane, ey, ex)
    return out.transpose(0, 2, 1).reshape(B, _COUT, H, W)


# speedup vs baseline: 2.1814x; 2.1814x over previous
import jax
import jax.numpy as jnp
import numpy as np
from jax.experimental import pallas as pl
from jax.experimental.pallas import tpu as pltpu

_B, _CIN, _H, _W = 4, 256, 56, 56
_COUT, _K = 256, 3
_KK = _K * _K
_HW = _H * _W
_TILE_P = 448
_NT = _HW // _TILE_P


def _dc_kernel(xt_ref, xb_ref, owr_ref, offb_ref, wr_ref, rw_ref, wcol_ref,
               yx_ref, out_ref, offs_ref):
    t = pl.program_id(1)
    x = xt_ref[0]

    @pl.when(t == 0)
    def _compute_offsets():
        wcol = wcol_ref[...]
        acc = jnp.zeros((_HW, 2 * _KK), jnp.float32) + offb_ref[...]
        for j in range(_KK):
            dy, dx = j // 3 - 1, j % 3 - 1
            s = dy * _W + dx
            if s > 0:
                xs = jnp.concatenate(
                    [x[s:], jnp.zeros((s, _CIN), jnp.float32)], axis=0)
            elif s < 0:
                xs = jnp.concatenate(
                    [jnp.zeros((-s, _CIN), jnp.float32), x[:_HW + s]], axis=0)
            else:
                xs = x
            if dx == 1:
                xs = jnp.where(wcol < _W - 1.5, xs, 0.0)
            elif dx == -1:
                xs = jnp.where(wcol > 0.5, xs, 0.0)
            acc = acc + jnp.dot(xs, owr_ref[j],
                                preferred_element_type=jnp.float32)
        offs_ref[...] = acc

    off_t = offs_ref[pl.ds(t * _TILE_P, _TILE_P), :]
    r = rw_ref[:, 0:1]
    w = rw_ref[:, 1:2]
    yq = yx_ref[0:1, :]
    xq = yx_ref[1:2, :]
    xb = xb_ref[0]
    one = jnp.bfloat16(1.0)
    zero = jnp.bfloat16(0.0)

    acc = jnp.zeros((_TILE_P, _COUT), jnp.float32)
    for kk in range(_KK):
        kh, kw = kk // 3, kk % 3
        py = r + (kh - 1) + off_t[:, 2 * kk:2 * kk + 1]
        px = w + (kw - 1) + off_t[:, 2 * kk + 1:2 * kk + 2]
        dy = (yq - py).astype(jnp.bfloat16)
        dx = (xq - px).astype(jnp.bfloat16)
        wy = jnp.maximum(zero, one - jnp.abs(dy))
        wx = jnp.maximum(zero, one - jnp.abs(dx))
        samp = jnp.dot(wy * wx, xb, preferred_element_type=jnp.float32)
        acc = acc + jnp.dot(samp, wr_ref[kk],
                            preferred_element_type=jnp.float32)
    out_ref[0] = acc


@jax.jit
def kernel(x, weight, off_w, off_b):
    B, C, H, W = x.shape
    xt = jnp.transpose(x, (0, 2, 3, 1)).reshape(B, _HW, C)
    owr = jnp.transpose(off_w, (2, 3, 1, 0)).reshape(_KK, C, 2 * _KK)
    offb2 = off_b.reshape(1, 2 * _KK)
    wr = jnp.transpose(weight.reshape(_COUT, C, _KK), (2, 1, 0))

    xb = xt.astype(jnp.bfloat16)
    pv = np.arange(_HW)
    rw = jnp.asarray(np.stack([pv // _W, pv % _W], axis=1), jnp.float32)
    wcol = jnp.asarray((pv % _W)[:, None], jnp.float32)
    yx = jnp.asarray(np.stack([pv // _W, pv % _W], axis=0), jnp.float32)

    out = pl.pallas_call(
        _dc_kernel,
        grid=(B, _NT),
        in_specs=[
            pl.BlockSpec((1, _HW, C), lambda b, t: (b, 0, 0)),
            pl.BlockSpec((1, _HW, C), lambda b, t: (b, 0, 0)),
            pl.BlockSpec((_KK, C, 2 * _KK), lambda b, t: (0, 0, 0)),
            pl.BlockSpec((1, 2 * _KK), lambda b, t: (0, 0)),
            pl.BlockSpec((_KK, C, _COUT), lambda b, t: (0, 0, 0)),
            pl.BlockSpec((_TILE_P, 2), lambda b, t: (t, 0)),
            pl.BlockSpec((_HW, 1), lambda b, t: (0, 0)),
            pl.BlockSpec((2, _HW), lambda b, t: (0, 0)),
        ],
        out_specs=pl.BlockSpec((1, _TILE_P, _COUT), lambda b, t: (b, t, 0)),
        out_shape=jax.ShapeDtypeStruct((B, _HW, _COUT), jnp.float32),
        scratch_shapes=[pltpu.VMEM((_HW, 2 * _KK), jnp.float32)],
        compiler_params=pltpu.CompilerParams(
            dimension_semantics=("parallel", "arbitrary"),
        ),
    )(xt, xb, owr, offb2, wr, rw, wcol, yx)
    return out.transpose(0, 2, 1).reshape(B, _COUT, H, W)


# speedup vs baseline: 2.6404x; 1.2104x over previous
import jax
import jax.numpy as jnp
import numpy as np
from jax.experimental import pallas as pl
from jax.experimental.pallas import tpu as pltpu

_B, _CIN, _H, _W = 4, 256, 56, 56
_COUT, _K = 256, 3
_KK = _K * _K
_HW = _H * _W
_TILE_P = 448
_NT = _HW // _TILE_P


def _dc_kernel(xt_ref, xb_ref, owr_ref, offb_ref, wr_ref, rw_ref, wcol_ref,
               yx_ref, out_ref, offs_ref):
    t = pl.program_id(1)
    x = xt_ref[0]

    @pl.when(t == 0)
    def _compute_offsets():
        wcol = wcol_ref[...]
        acc = jnp.zeros((_HW, 2 * _KK), jnp.float32) + offb_ref[...]
        for j in range(_KK):
            dy, dx = j // 3 - 1, j % 3 - 1
            s = dy * _W + dx
            if s > 0:
                xs = jnp.concatenate(
                    [x[s:], jnp.zeros((s, _CIN), jnp.float32)], axis=0)
            elif s < 0:
                xs = jnp.concatenate(
                    [jnp.zeros((-s, _CIN), jnp.float32), x[:_HW + s]], axis=0)
            else:
                xs = x
            if dx == 1:
                xs = jnp.where(wcol < _W - 1.5, xs, 0.0)
            elif dx == -1:
                xs = jnp.where(wcol > 0.5, xs, 0.0)
            acc = acc + jnp.dot(xs, owr_ref[j],
                                preferred_element_type=jnp.float32)
        offs_ref[...] = acc

    off_t = offs_ref[pl.ds(t * _TILE_P, _TILE_P), :]
    r = rw_ref[:, 0:1]
    w = rw_ref[:, 1:2]
    yq = yx_ref[0:1, :]
    xq = yx_ref[1:2, :]
    xb = xb_ref[0]
    one = jnp.bfloat16(1.0)
    zero = jnp.bfloat16(0.0)

    g = (yq - r).astype(jnp.bfloat16)
    h = (xq - w).astype(jnp.bfloat16)

    acc = jnp.zeros((_TILE_P, _COUT), jnp.float32)
    for kk in range(_KK):
        kh, kw = kk // 3, kk % 3
        cy = ((kh - 1) + off_t[:, 2 * kk:2 * kk + 1]).astype(jnp.bfloat16)
        cx = ((kw - 1) + off_t[:, 2 * kk + 1:2 * kk + 2]).astype(jnp.bfloat16)
        wy = jnp.maximum(zero, one - jnp.abs(g - cy))
        wx = jnp.maximum(zero, one - jnp.abs(h - cx))
        samp = jnp.dot(wy * wx, xb, preferred_element_type=jnp.float32)
        acc = acc + jnp.dot(samp, wr_ref[kk],
                            preferred_element_type=jnp.float32)
    out_ref[0] = acc


@jax.jit
def kernel(x, weight, off_w, off_b):
    B, C, H, W = x.shape
    xt = jnp.transpose(x, (0, 2, 3, 1)).reshape(B, _HW, C)
    owr = jnp.transpose(off_w, (2, 3, 1, 0)).reshape(_KK, C, 2 * _KK)
    offb2 = off_b.reshape(1, 2 * _KK)
    wr = jnp.transpose(weight.reshape(_COUT, C, _KK), (2, 1, 0))

    xb = xt.astype(jnp.bfloat16)
    pv = np.arange(_HW)
    rw = jnp.asarray(np.stack([pv // _W, pv % _W], axis=1), jnp.float32)
    wcol = jnp.asarray((pv % _W)[:, None], jnp.float32)
    yx = jnp.asarray(np.stack([pv // _W, pv % _W], axis=0), jnp.float32)

    out = pl.pallas_call(
        _dc_kernel,
        grid=(B, _NT),
        in_specs=[
            pl.BlockSpec((1, _HW, C), lambda b, t: (b, 0, 0)),
            pl.BlockSpec((1, _HW, C), lambda b, t: (b, 0, 0)),
            pl.BlockSpec((_KK, C, 2 * _KK), lambda b, t: (0, 0, 0)),
            pl.BlockSpec((1, 2 * _KK), lambda b, t: (0, 0)),
            pl.BlockSpec((_KK, C, _COUT), lambda b, t: (0, 0, 0)),
            pl.BlockSpec((_TILE_P, 2), lambda b, t: (t, 0)),
            pl.BlockSpec((_HW, 1), lambda b, t: (0, 0)),
            pl.BlockSpec((2, _HW), lambda b, t: (0, 0)),
        ],
        out_specs=pl.BlockSpec((1, _TILE_P, _COUT), lambda b, t: (b, t, 0)),
        out_shape=jax.ShapeDtypeStruct((B, _HW, _COUT), jnp.float32),
        scratch_shapes=[pltpu.VMEM((_HW, 2 * _KK), jnp.float32)],
        compiler_params=pltpu.CompilerParams(
            dimension_semantics=("parallel", "arbitrary"),
        ),
    )(xt, xb, owr, offb2, wr, rw, wcol, yx)
    return out.transpose(0, 2, 1).reshape(B, _COUT, H, W)


# speedup vs baseline: 2.8217x; 1.0687x over previous
import jax
import jax.numpy as jnp
import numpy as np
from jax.experimental import pallas as pl
from jax.experimental.pallas import tpu as pltpu

_B, _CIN, _H, _W = 4, 256, 56, 56
_COUT, _K = 256, 3
_KK = _K * _K
_HW = _H * _W
_TILE_P = 784
_NT = _HW // _TILE_P


def _dc_kernel(xt_ref, xb_ref, owr_ref, offb_ref, wr_ref, rw_ref, wcol_ref,
               yx_ref, out_ref, offs_ref):
    t = pl.program_id(1)
    x = xt_ref[0]

    @pl.when(t == 0)
    def _compute_offsets():
        wcol = wcol_ref[...]
        acc = jnp.zeros((_HW, 2 * _KK), jnp.float32) + offb_ref[...]
        for j in range(_KK):
            dy, dx = j // 3 - 1, j % 3 - 1
            s = dy * _W + dx
            if s > 0:
                xs = jnp.concatenate(
                    [x[s:], jnp.zeros((s, _CIN), jnp.float32)], axis=0)
            elif s < 0:
                xs = jnp.concatenate(
                    [jnp.zeros((-s, _CIN), jnp.float32), x[:_HW + s]], axis=0)
            else:
                xs = x
            if dx == 1:
                xs = jnp.where(wcol < _W - 1.5, xs, 0.0)
            elif dx == -1:
                xs = jnp.where(wcol > 0.5, xs, 0.0)
            acc = acc + jnp.dot(xs, owr_ref[j],
                                preferred_element_type=jnp.float32)
        offs_ref[...] = acc

    off_t = offs_ref[pl.ds(t * _TILE_P, _TILE_P), :]
    r = rw_ref[:, 0:1]
    w = rw_ref[:, 1:2]
    yq = yx_ref[0:1, :]
    xq = yx_ref[1:2, :]
    xb = xb_ref[0]
    one = jnp.bfloat16(1.0)
    zero = jnp.bfloat16(0.0)

    g = (yq - r).astype(jnp.bfloat16)
    h = (xq - w).astype(jnp.bfloat16)

    samps = []
    for kk in range(_KK):
        kh, kw = kk // 3, kk % 3
        cy = ((kh - 1) + off_t[:, 2 * kk:2 * kk + 1]).astype(jnp.bfloat16)
        cx = ((kw - 1) + off_t[:, 2 * kk + 1:2 * kk + 2]).astype(jnp.bfloat16)
        wy = jnp.maximum(zero, one - jnp.abs(g - cy))
        wx = jnp.maximum(zero, one - jnp.abs(h - cx))
        samps.append(jnp.dot(wy * wx, xb, preferred_element_type=jnp.float32))
    samp_all = jnp.concatenate(samps, axis=1)
    out_ref[0] = jnp.dot(samp_all, wr_ref[...],
                         preferred_element_type=jnp.float32)


@jax.jit
def kernel(x, weight, off_w, off_b):
    B, C, H, W = x.shape
    xt = jnp.transpose(x, (0, 2, 3, 1)).reshape(B, _HW, C)
    owr = jnp.transpose(off_w, (2, 3, 1, 0)).reshape(_KK, C, 2 * _KK)
    offb2 = off_b.reshape(1, 2 * _KK)
    wr = jnp.transpose(weight.reshape(_COUT, C, _KK), (2, 1, 0)).reshape(
        _KK * C, _COUT)

    xb = xt.astype(jnp.bfloat16)
    pv = np.arange(_HW)
    rw = jnp.asarray(np.stack([pv // _W, pv % _W], axis=1), jnp.float32)
    wcol = jnp.asarray((pv % _W)[:, None], jnp.float32)
    yx = jnp.asarray(np.stack([pv // _W, pv % _W], axis=0), jnp.float32)

    out = pl.pallas_call(
        _dc_kernel,
        grid=(B, _NT),
        in_specs=[
            pl.BlockSpec((1, _HW, C), lambda b, t: (b, 0, 0)),
            pl.BlockSpec((1, _HW, C), lambda b, t: (b, 0, 0)),
            pl.BlockSpec((_KK, C, 2 * _KK), lambda b, t: (0, 0, 0)),
            pl.BlockSpec((1, 2 * _KK), lambda b, t: (0, 0)),
            pl.BlockSpec((_KK * C, _COUT), lambda b, t: (0, 0)),
            pl.BlockSpec((_TILE_P, 2), lambda b, t: (t, 0)),
            pl.BlockSpec((_HW, 1), lambda b, t: (0, 0)),
            pl.BlockSpec((2, _HW), lambda b, t: (0, 0)),
        ],
        out_specs=pl.BlockSpec((1, _TILE_P, _COUT), lambda b, t: (b, t, 0)),
        out_shape=jax.ShapeDtypeStruct((B, _HW, _COUT), jnp.float32),
        scratch_shapes=[pltpu.VMEM((_HW, 2 * _KK), jnp.float32)],
        compiler_params=pltpu.CompilerParams(
            dimension_semantics=("parallel", "arbitrary"),
        ),
    )(xt, xb, owr, offb2, wr, rw, wcol, yx)
    return out.transpose(0, 2, 1).reshape(B, _COUT, H, W)
